# baseline (speedup 1.0000x reference)
import functools

import jax
import jax.numpy as jnp
from jax.experimental import pallas as pl
from jax.experimental.pallas import tpu as pltpu

_SEQ_PAD = 256


def _body(user_ref, items_ref, q_ref, emb_ref, out_ref):
    del user_ref
    items_v = items_ref[0, :]
    n_items = q_ref.shape[0]
    col = jax.lax.broadcasted_iota(jnp.int32, (_SEQ_PAD, n_items), 1)
    onehot = (col == items_v[:, None]).astype(jnp.float32)
    qf = q_ref[...].astype(jnp.float32)
    q_rows = jnp.dot(onehot, qf, preferred_element_type=jnp.float32)
    emb = emb_ref[0]
    out_ref[...] = jnp.dot(q_rows, emb, preferred_element_type=jnp.float32)


def kernel(user, Q_matrix, items, skill_embedding):
    seq_len = items.shape[0]
    n_items, skill_num = Q_matrix.shape
    k_hidden = skill_embedding.shape[2]
    user_arr = jnp.asarray(user, jnp.int32).reshape(1)
    items_pad = jnp.zeros((1, _SEQ_PAD), jnp.int32).at[0, :seq_len].set(
        items.astype(jnp.int32))

    grid_spec = pltpu.PrefetchScalarGridSpec(
        num_scalar_prefetch=1,
        grid=(1,),
        in_specs=[
            pl.BlockSpec((1, _SEQ_PAD), lambda i, u: (0, 0)),
            pl.BlockSpec((n_items, skill_num), lambda i, u: (0, 0)),
            pl.BlockSpec((1, skill_num, k_hidden), lambda i, u: (u[0], 0, 0)),
        ],
        out_specs=pl.BlockSpec((_SEQ_PAD, k_hidden), lambda i, u: (0, 0)),
    )
    out = pl.pallas_call(
        _body,
        grid_spec=grid_spec,
        out_shape=jax.ShapeDtypeStruct((_SEQ_PAD, k_hidden), jnp.float32),
    )(user_arr, items_pad, Q_matrix, skill_embedding)
    return out[:seq_len]
